# baseline (speedup 1.0000x reference)
import functools

import jax
import jax.numpy as jnp
from jax import lax
from jax.experimental import pallas as pl
from jax.experimental.pallas import tpu as pltpu
from jax.experimental.pallas import tpu_sc as plsc

_S = 64.0
_M = 0.7
_BR = 128
_NW = 32
_L = 16


def _sc_gather_body(lab_hbm, x1d_hbm, out_hbm, lab_v, idx_v, val_v, sem):
    b = lab_hbm.shape[0]
    c = x1d_hbm.shape[0] // b
    bpw = b // _NW
    wid = lax.axis_index("s") * 2 + lax.axis_index("c")
    base = wid * bpw
    pltpu.sync_copy(lab_hbm.at[pl.ds(base, bpw)], lab_v)
    for j in range(bpw // _L):
        row = base + j * _L + lax.iota(jnp.int32, _L)
        idx_v[pl.ds(j * _L, _L)] = row * c + lab_v[pl.ds(j * _L, _L)]
    pltpu.async_copy(x1d_hbm.at[idx_v], val_v, sem).wait()
    pltpu.sync_copy(val_v, out_hbm.at[pl.ds(base, bpw)])


def _tc_body(lab_ref, t_ref, x_ref, diff_ref, theta_ref):
    x = x_ref[...]
    lab = lab_ref[...]
    br, c = x.shape

    t = jnp.clip(t_ref[...], -1.0, 1.0)
    y = jnp.abs(t)
    p = jnp.float32(-0.0012624911)
    for coef in (0.0066700901, -0.0170881256, 0.0308918810,
                 -0.0501743046, 0.0889789874, -0.2145988016,
                 1.5707963050):
        p = p * y + jnp.float32(coef)
    r = jnp.sqrt(jnp.maximum(1.0 - y, 0.0)) * p
    theta = jnp.where(t >= 0.0, r, jnp.pi - r)
    tpm = jnp.pi * jnp.exp(_M * jnp.log(theta * (1.0 / jnp.pi)))
    ftl = jnp.cos(tpm)
    theta_ref[...] = tpm - theta

    a = x[:, : c - 1]
    b = x[:, 1:]
    cols = lax.broadcasted_iota(jnp.int32, (br, c - 1), 1)
    sel = jnp.where(cols < lab, a, b)
    diff_ref[...] = sel * _S - ftl * _S


@jax.jit
def kernel(logits, labels):
    b, c = logits.shape
    x1d = logits.reshape(b * c)
    bpw = b // _NW

    sc_gather = functools.partial(
        pl.kernel,
        mesh=plsc.VectorSubcoreMesh(core_axis_name="c", subcore_axis_name="s"),
        out_type=jax.ShapeDtypeStruct((b,), jnp.float32),
        scratch_types=[
            pltpu.VMEM((bpw,), jnp.int32),
            pltpu.VMEM((bpw,), jnp.int32),
            pltpu.VMEM((bpw,), jnp.float32),
            pltpu.SemaphoreType.DMA,
        ],
    )(_sc_gather_body)
    t = sc_gather(labels, x1d)

    lab2 = labels.reshape(b, 1)
    t2 = t.reshape(b, 1)
    grid = b // _BR
    diff, theta_m = pl.pallas_call(
        _tc_body,
        grid=(grid,),
        in_specs=[
            pl.BlockSpec((_BR, 1), lambda i: (i, 0)),
            pl.BlockSpec((_BR, 1), lambda i: (i, 0)),
            pl.BlockSpec((_BR, c), lambda i: (i, 0)),
        ],
        out_specs=[
            pl.BlockSpec((_BR, c - 1), lambda i: (i, 0)),
            pl.BlockSpec((_BR, 1), lambda i: (i, 0)),
        ],
        out_shape=[
            jax.ShapeDtypeStruct((b, c - 1), jnp.float32),
            jax.ShapeDtypeStruct((b, 1), jnp.float32),
        ],
    )(lab2, t2, logits)
    return diff, theta_m.reshape(b)


# speedup vs baseline: 1.4268x; 1.4268x over previous
import functools

import jax
import jax.numpy as jnp
from jax import lax
from jax.experimental import pallas as pl
from jax.experimental.pallas import tpu as pltpu

_S = 64.0
_M = 0.7
_BR = 256


def _tc_body(lab_ref, x_ref, diff_ref, theta_ref):
    x = x_ref[...]
    lab = lab_ref[...]
    br, c = x.shape

    cols_full = lax.broadcasted_iota(jnp.int32, (br, c), 1)
    t = jnp.sum(jnp.where(cols_full == lab, x, 0.0), axis=1, keepdims=True)

    t = jnp.clip(t, -1.0, 1.0)
    y = jnp.abs(t)
    p = jnp.float32(-0.0012624911)
    for coef in (0.0066700901, -0.0170881256, 0.0308918810,
                 -0.0501743046, 0.0889789874, -0.2145988016,
                 1.5707963050):
        p = p * y + jnp.float32(coef)
    r = jnp.sqrt(jnp.maximum(1.0 - y, 0.0)) * p
    theta = jnp.where(t >= 0.0, r, jnp.pi - r)
    tpm = jnp.pi * jnp.exp(_M * jnp.log(theta * (1.0 / jnp.pi)))
    ftl = jnp.cos(tpm)
    theta_ref[...] = tpm - theta

    a = x[:, : c - 1]
    b = x[:, 1:]
    cols = cols_full[:, : c - 1]
    sel = jnp.where(cols < lab, a, b)
    diff_ref[...] = sel * _S - ftl * _S


@jax.jit
def kernel(logits, labels):
    b, c = logits.shape
    lab2 = labels.reshape(b, 1)
    grid = b // _BR
    diff, theta_m = pl.pallas_call(
        _tc_body,
        grid=(grid,),
        in_specs=[
            pl.BlockSpec((_BR, 1), lambda i: (i, 0)),
            pl.BlockSpec((_BR, c), lambda i: (i, 0)),
        ],
        out_specs=[
            pl.BlockSpec((_BR, c - 1), lambda i: (i, 0)),
            pl.BlockSpec((_BR, 1), lambda i: (i, 0)),
        ],
        out_shape=[
            jax.ShapeDtypeStruct((b, c - 1), jnp.float32),
            jax.ShapeDtypeStruct((b, 1), jnp.float32),
        ],
    )(lab2, logits)
    return diff, theta_m.reshape(b)
